# baseline (speedup 1.0000x reference)
import functools

import jax
import jax.numpy as jnp
from jax import lax
from jax.experimental import pallas as pl
from jax.experimental.pallas import tpu as pltpu
from jax.experimental.pallas import tpu_sc as plsc

N = 10000
E = 320000
D = 128
DH = D // 2
DE = 16
DDG = 8

NC = 2
NS = 16
NW = NC * NS
CH = 128
EPW = E // NW
NFULL_P = EPW // CH
TAIL_P = EPW - NFULL_P * CH
EPT = E // NS
NFULL_E = EPT // CH
TAIL_E = EPT - NFULL_E * CH
NPAD = 10240
RPT = NPAD // NS
ZR = 320
TAIL = N - (NS - 1) * RPT - ZR
NV = N - (NS - 1) * RPT


@functools.lru_cache(maxsize=None)
def _mesh():
    return plsc.VectorSubcoreMesh(
        core_axis_name="c", subcore_axis_name="s",
        num_cores=NC, num_subcores=NS)


def _leaky(v):
    return jnp.where(v >= 0, v, 0.01 * v)


def _deg_body(dst_hbm, ones_hbm, z8_hbm, out_dg,
              dsti, ones_buf, st8, acc_dg, sem0, sem1):
    c = lax.axis_index("c")
    s = lax.axis_index("s")
    base = (c * NS + s) * EPW

    pltpu.sync_copy(z8_hbm, st8)
    pltpu.sync_copy(st8, acc_dg.at[pl.ds(s * RPT, RPT)])
    pltpu.sync_copy(ones_hbm, ones_buf)
    pltpu.sync_copy(dst_hbm.at[pl.ds(base, EPW)], dsti)
    plsc.subcore_barrier()

    def body(i, carry):
        for b, sem in enumerate((sem0, sem1)):
            jj = 2 * i + b
            pltpu.async_copy(
                ones_buf, acc_dg.at[dsti.at[pl.ds(jj * CH, CH)]], sem,
                add=True)
        for b, sem in enumerate((sem0, sem1)):
            jj = 2 * i + b
            pltpu.make_async_copy(
                ones_buf, acc_dg.at[dsti.at[pl.ds(jj * CH, CH)]], sem).wait()
        return carry

    lax.fori_loop(0, NFULL_P // 2, body, 0)
    pltpu.sync_copy(ones_buf.at[pl.ds(0, TAIL_P)],
                    acc_dg.at[dsti.at[pl.ds(NFULL_P * CH, TAIL_P)]], add=True)
    plsc.subcore_barrier()

    pltpu.sync_copy(acc_dg.at[pl.ds(s * RPT, RPT)], st8)

    @pl.when(s < NS - 1)
    def _():
        pltpu.sync_copy(
            st8, out_dg.at[pl.ds(s * RPT, RPT), pl.ds(c * DDG, DDG)])

    @pl.when(s == NS - 1)
    def _():
        pltpu.sync_copy(
            st8.at[pl.ds(0, NV)],
            out_dg.at[pl.ds((NS - 1) * RPT, NV), pl.ds(c * DDG, DDG)])


@functools.lru_cache(maxsize=None)
def _deg_pass():
    return pl.kernel(
        _deg_body,
        out_type=jax.ShapeDtypeStruct((N, 2 * DDG), jnp.float32),
        mesh=_mesh(),
        scratch_types=[
            pltpu.VMEM((EPW,), jnp.int32),
            pltpu.VMEM((CH, DDG), jnp.float32),
            pltpu.VMEM((RPT, DDG), jnp.float32),
            pltpu.VMEM_SHARED((NPAD, DDG), jnp.float32),
            pltpu.SemaphoreType.DMA,
            pltpu.SemaphoreType.DMA,
        ],
        compiler_params=pltpu.CompilerParams(use_tc_tiling_on_sc=False),
    )


def _ea_body(ea_hbm, dst_hbm, z16_hbm, dep_hbm, out_ea,
             dsti, ea_buf0, ea_buf1, st16, acc_ea, sem0, sem1):
    del dep_hbm
    c = lax.axis_index("c")
    s = lax.axis_index("s")
    base = (c * NS + s) * EPW

    pltpu.sync_copy(z16_hbm, st16)
    pltpu.sync_copy(st16, acc_ea.at[pl.ds(s * RPT, RPT)])
    pltpu.sync_copy(dst_hbm.at[pl.ds(base, EPW)], dsti)
    plsc.subcore_barrier()

    pltpu.async_copy(ea_hbm.at[pl.ds(base, CH)], ea_buf0, sem0)
    pltpu.async_copy(ea_hbm.at[pl.ds(base + CH, CH)], ea_buf1, sem1)

    def body(i, carry):
        for b, (buf, sem) in enumerate(((ea_buf0, sem0), (ea_buf1, sem1))):
            jj = 2 * i + b
            pltpu.make_async_copy(
                ea_hbm.at[pl.ds(base + jj * CH, CH)], buf, sem).wait()
            pltpu.sync_copy(buf, acc_ea.at[dsti.at[pl.ds(jj * CH, CH)]],
                            add=True)

            @pl.when(jj + 2 < NFULL_P)
            def _():
                pltpu.async_copy(
                    ea_hbm.at[pl.ds(base + (jj + 2) * CH, CH)], buf, sem)
        return carry

    lax.fori_loop(0, NFULL_P // 2, body, 0)

    pltpu.sync_copy(ea_hbm.at[pl.ds(base + NFULL_P * CH, TAIL_P)],
                    ea_buf0.at[pl.ds(0, TAIL_P)])
    pltpu.sync_copy(ea_buf0.at[pl.ds(0, TAIL_P)],
                    acc_ea.at[dsti.at[pl.ds(NFULL_P * CH, TAIL_P)]], add=True)
    plsc.subcore_barrier()

    pltpu.sync_copy(acc_ea.at[pl.ds(s * RPT, RPT)], st16)

    @pl.when(s < NS - 1)
    def _():
        pltpu.sync_copy(
            st16, out_ea.at[pl.ds(s * RPT, RPT), pl.ds(c * DE, DE)])

    @pl.when(s == NS - 1)
    def _():
        pltpu.sync_copy(
            st16.at[pl.ds(0, NV)],
            out_ea.at[pl.ds((NS - 1) * RPT, NV), pl.ds(c * DE, DE)])


@functools.lru_cache(maxsize=None)
def _ea_pass():
    return pl.kernel(
        _ea_body,
        out_type=jax.ShapeDtypeStruct((N, 2 * DE), jnp.float32),
        mesh=_mesh(),
        scratch_types=[
            pltpu.VMEM((EPW,), jnp.int32),
            pltpu.VMEM((CH, DE), jnp.float32),
            pltpu.VMEM((CH, DE), jnp.float32),
            pltpu.VMEM((RPT, DE), jnp.float32),
            pltpu.VMEM_SHARED((NPAD, DE), jnp.float32),
            pltpu.SemaphoreType.DMA,
            pltpu.SemaphoreType.DMA,
        ],
        compiler_params=pltpu.CompilerParams(use_tc_tiling_on_sc=False),
    )


def _edge_body(t_hbm, src2_hbm, dst_hbm, zero_hbm, out_p,
               srci, dsti, rows0, rows1, st_buf, acc, sem0, sem1):
    c = lax.axis_index("c")
    s = lax.axis_index("s")

    pltpu.sync_copy(zero_hbm, st_buf)
    pltpu.sync_copy(st_buf, acc.at[pl.ds(s * RPT, ZR)])
    pltpu.sync_copy(st_buf, acc.at[pl.ds(s * RPT + ZR, ZR)])
    pltpu.sync_copy(src2_hbm.at[c, pl.ds(s * EPT, EPT)], srci)
    pltpu.sync_copy(dst_hbm.at[pl.ds(s * EPT, EPT)], dsti)
    plsc.subcore_barrier()

    pltpu.async_copy(t_hbm.at[srci.at[pl.ds(0, CH)]], rows0, sem0)
    pltpu.async_copy(t_hbm.at[srci.at[pl.ds(CH, CH)]], rows1, sem1)

    def body(i, carry):
        for b, (rows, sem) in enumerate(((rows0, sem0), (rows1, sem1))):
            jj = 2 * i + b
            pltpu.make_async_copy(
                t_hbm.at[srci.at[pl.ds(jj * CH, CH)]], rows, sem).wait()
            pltpu.sync_copy(rows, acc.at[dsti.at[pl.ds(jj * CH, CH)]],
                            add=True)

            @pl.when(jj + 2 < NFULL_E)
            def _():
                pltpu.async_copy(
                    t_hbm.at[srci.at[pl.ds((jj + 2) * CH, CH)]], rows, sem)
        return carry

    lax.fori_loop(0, NFULL_E // 2, body, 0)

    pltpu.async_copy(
        t_hbm.at[srci.at[pl.ds(NFULL_E * CH, TAIL_E)]],
        rows0.at[pl.ds(0, TAIL_E)], sem0)
    pltpu.make_async_copy(
        t_hbm.at[srci.at[pl.ds(NFULL_E * CH, TAIL_E)]],
        rows0.at[pl.ds(0, TAIL_E)], sem0).wait()
    pltpu.sync_copy(rows0.at[pl.ds(0, TAIL_E)],
                    acc.at[dsti.at[pl.ds(NFULL_E * CH, TAIL_E)]], add=True)
    plsc.subcore_barrier()

    pltpu.sync_copy(acc.at[pl.ds(s * RPT, ZR)], st_buf)
    pltpu.sync_copy(
        st_buf, out_p.at[pl.ds(s * RPT, ZR), pl.ds(c * DH, DH)])
    pltpu.sync_copy(acc.at[pl.ds(s * RPT + ZR, ZR)], st_buf)

    @pl.when(s < NS - 1)
    def _():
        pltpu.sync_copy(
            st_buf, out_p.at[pl.ds(s * RPT + ZR, ZR), pl.ds(c * DH, DH)])

    @pl.when(s == NS - 1)
    def _():
        pltpu.sync_copy(
            st_buf.at[pl.ds(0, TAIL)],
            out_p.at[pl.ds((NS - 1) * RPT + ZR, TAIL), pl.ds(c * DH, DH)])


@functools.lru_cache(maxsize=None)
def _edge_pass():
    return pl.kernel(
        _edge_body,
        out_type=jax.ShapeDtypeStruct((N, D), jnp.float32),
        mesh=_mesh(),
        scratch_types=[
            pltpu.VMEM((EPT,), jnp.int32),
            pltpu.VMEM((EPT,), jnp.int32),
            pltpu.VMEM((CH, DH), jnp.float32),
            pltpu.VMEM((CH, DH), jnp.float32),
            pltpu.VMEM((ZR, DH), jnp.float32),
            pltpu.VMEM_SHARED((NPAD, DH), jnp.float32),
            pltpu.SemaphoreType.DMA,
            pltpu.SemaphoreType.DMA,
        ],
        compiler_params=pltpu.CompilerParams(use_tc_tiling_on_sc=False),
    )


_BLK = 1000
_GRID = N // _BLK


def _r_from_dg(dg):
    deg = dg[:, 0:1] + dg[:, DDG:DDG + 1]
    return lax.rsqrt(jnp.maximum(deg, 1.0))


def _tc_pre_body(x_ref, w_ref, dg_ref, t_ref):
    r = _r_from_dg(dg_ref[...])
    t_ref[...] = jnp.dot(x_ref[...], w_ref[...],
                         preferred_element_type=jnp.float32) * r


def _tc_mid_body(p_ref, ea_ref, dg_ref, w_ref, we1_ref, b1_ref, we2_ref,
                 b2_ref, t_ref, base2_ref):
    r = _r_from_dg(dg_ref[...])
    ea = ea_ref[:, 0:DE] + ea_ref[:, DE:2 * DE]
    base1 = jnp.dot(ea, we1_ref[...],
                    preferred_element_type=jnp.float32) + b1_ref[...]
    h = _leaky(r * p_ref[...] + base1)
    t_ref[...] = jnp.dot(h, w_ref[...], preferred_element_type=jnp.float32) * r
    base2_ref[...] = jnp.dot(ea, we2_ref[...],
                             preferred_element_type=jnp.float32) + b2_ref[...]


def _tc_post_body(p_ref, base_ref, dg_ref, out_ref):
    r = _r_from_dg(dg_ref[...])
    out_ref[...] = _leaky(r * p_ref[...] + base_ref[...])


def _idx_body(ei_ref, out_ref):
    sv = ei_ref[0]
    out_ref[0] = 2 * sv
    out_ref[1] = 2 * sv + 1


_EBLK = E // 10

_tc_idx = pl.pallas_call(
    _idx_body,
    grid=(10,),
    in_specs=[pl.BlockSpec((2, _EBLK), lambda i: (0, i))],
    out_specs=pl.BlockSpec((8, _EBLK), lambda i: (0, i)),
    out_shape=jax.ShapeDtypeStruct((8, E), jnp.int32),
)


def _row_spec(width):
    return pl.BlockSpec((_BLK, width), lambda i: (i, 0))


def _full_spec(a, b):
    return pl.BlockSpec((a, b), lambda i: (0, 0))


_f32 = jnp.float32
_sds = jax.ShapeDtypeStruct

_tc_pre = pl.pallas_call(
    _tc_pre_body,
    grid=(_GRID,),
    in_specs=[_row_spec(D), _full_spec(D, D), _row_spec(2 * DDG)],
    out_specs=_row_spec(D),
    out_shape=_sds((N, D), _f32),
)

_tc_mid = pl.pallas_call(
    _tc_mid_body,
    grid=(_GRID,),
    in_specs=[_row_spec(D), _row_spec(2 * DE), _row_spec(2 * DDG),
              _full_spec(D, D), _full_spec(DE, D), _full_spec(1, D),
              _full_spec(DE, D), _full_spec(1, D)],
    out_specs=(_row_spec(D), _row_spec(D)),
    out_shape=(_sds((N, D), _f32), _sds((N, D), _f32)),
)

_tc_post = pl.pallas_call(
    _tc_post_body,
    grid=(_GRID,),
    in_specs=[_row_spec(D), _row_spec(D), _row_spec(2 * DDG)],
    out_specs=_row_spec(D),
    out_shape=_sds((N, D), _f32),
)


def kernel(x, edge_index, edge_attr, W1, We1, b1, W2, We2, b2):
    dst = edge_index[1]
    src2 = _tc_idx(edge_index)
    ones8 = jnp.ones((CH, DDG), _f32)
    z16 = jnp.zeros((RPT, DE), _f32)
    z8 = jnp.zeros((RPT, DDG), _f32)
    z_dh = jnp.zeros((ZR, DH), _f32)

    dg = _deg_pass()(dst, ones8, z8)
    t1 = _tc_pre(x, W1, dg)
    p1 = _edge_pass()(t1.reshape(2 * N, DH), src2, dst, z_dh)
    ea = _ea_pass()(edge_attr, dst, z16, p1)
    t2, base2 = _tc_mid(p1, ea, dg, W2, We1, b1.reshape(1, D),
                        We2, b2.reshape(1, D))
    p2 = _edge_pass()(t2.reshape(2 * N, DH), src2, dst, z_dh)
    return _tc_post(p2, base2, dg)


# speedup vs baseline: 1.0068x; 1.0068x over previous
import functools

import jax
import jax.numpy as jnp
from jax import lax
from jax.experimental import pallas as pl
from jax.experimental.pallas import tpu as pltpu
from jax.experimental.pallas import tpu_sc as plsc

N = 10000
E = 320000
D = 128
DH = D // 2
DE = 16
DDG = 8

NC = 2
NS = 16
NW = NC * NS
CH = 128
EPW = E // NW
NFULL_P = EPW // CH
TAIL_P = EPW - NFULL_P * CH
EPT = E // NS
NFULL_E = EPT // CH
TAIL_E = EPT - NFULL_E * CH
NPAD = 10240
RPT = NPAD // NS
ZR = 320
TAIL = N - (NS - 1) * RPT - ZR
NV = N - (NS - 1) * RPT


@functools.lru_cache(maxsize=None)
def _mesh():
    return plsc.VectorSubcoreMesh(
        core_axis_name="c", subcore_axis_name="s",
        num_cores=NC, num_subcores=NS)


def _leaky(v):
    return jnp.where(v >= 0, v, 0.01 * v)


def _deg_body(dst_hbm, ones_hbm, z8_hbm, out_dg,
              dsti, ones_buf, st8, acc_dg, sem0, sem1):
    c = lax.axis_index("c")
    s = lax.axis_index("s")
    base = (c * NS + s) * EPW

    pltpu.sync_copy(z8_hbm, st8)
    pltpu.sync_copy(st8, acc_dg.at[pl.ds(s * RPT, RPT)])
    pltpu.sync_copy(ones_hbm, ones_buf)
    pltpu.sync_copy(dst_hbm.at[pl.ds(base, EPW)], dsti)
    plsc.subcore_barrier()

    def body(i, carry):
        for b, sem in enumerate((sem0, sem1)):
            jj = 2 * i + b
            pltpu.async_copy(
                ones_buf, acc_dg.at[dsti.at[pl.ds(jj * CH, CH)]], sem,
                add=True)
        for b, sem in enumerate((sem0, sem1)):
            jj = 2 * i + b
            pltpu.make_async_copy(
                ones_buf, acc_dg.at[dsti.at[pl.ds(jj * CH, CH)]], sem).wait()
        return carry

    lax.fori_loop(0, NFULL_P // 2, body, 0)
    pltpu.sync_copy(ones_buf.at[pl.ds(0, TAIL_P)],
                    acc_dg.at[dsti.at[pl.ds(NFULL_P * CH, TAIL_P)]], add=True)
    plsc.subcore_barrier()

    pltpu.sync_copy(acc_dg.at[pl.ds(s * RPT, RPT)], st8)

    @pl.when(s < NS - 1)
    def _():
        pltpu.sync_copy(
            st8, out_dg.at[pl.ds(s * RPT, RPT), pl.ds(c * DDG, DDG)])

    @pl.when(s == NS - 1)
    def _():
        pltpu.sync_copy(
            st8.at[pl.ds(0, NV)],
            out_dg.at[pl.ds((NS - 1) * RPT, NV), pl.ds(c * DDG, DDG)])


@functools.lru_cache(maxsize=None)
def _deg_pass():
    return pl.kernel(
        _deg_body,
        out_type=jax.ShapeDtypeStruct((N, 2 * DDG), jnp.float32),
        mesh=_mesh(),
        scratch_types=[
            pltpu.VMEM((EPW,), jnp.int32),
            pltpu.VMEM((CH, DDG), jnp.float32),
            pltpu.VMEM((RPT, DDG), jnp.float32),
            pltpu.VMEM_SHARED((NPAD, DDG), jnp.float32),
            pltpu.SemaphoreType.DMA,
            pltpu.SemaphoreType.DMA,
        ],
        compiler_params=pltpu.CompilerParams(use_tc_tiling_on_sc=False),
    )


def _ea_body(ea_hbm, dst_hbm, z16_hbm, dep_hbm, out_ea,
             dsti, ea_buf0, ea_buf1, st16, acc_ea, sem0, sem1):
    del dep_hbm
    c = lax.axis_index("c")
    s = lax.axis_index("s")
    base = (c * NS + s) * EPW

    pltpu.sync_copy(z16_hbm, st16)
    pltpu.sync_copy(st16, acc_ea.at[pl.ds(s * RPT, RPT)])
    pltpu.sync_copy(dst_hbm.at[pl.ds(base, EPW)], dsti)
    plsc.subcore_barrier()

    pltpu.async_copy(ea_hbm.at[pl.ds(base, CH)], ea_buf0, sem0)
    pltpu.async_copy(ea_hbm.at[pl.ds(base + CH, CH)], ea_buf1, sem1)

    def body(i, carry):
        for b, (buf, sem) in enumerate(((ea_buf0, sem0), (ea_buf1, sem1))):
            jj = 2 * i + b
            pltpu.make_async_copy(
                ea_hbm.at[pl.ds(base + jj * CH, CH)], buf, sem).wait()
            pltpu.sync_copy(buf, acc_ea.at[dsti.at[pl.ds(jj * CH, CH)]],
                            add=True)

            @pl.when(jj + 2 < NFULL_P)
            def _():
                pltpu.async_copy(
                    ea_hbm.at[pl.ds(base + (jj + 2) * CH, CH)], buf, sem)
        return carry

    lax.fori_loop(0, NFULL_P // 2, body, 0)

    pltpu.sync_copy(ea_hbm.at[pl.ds(base + NFULL_P * CH, TAIL_P)],
                    ea_buf0.at[pl.ds(0, TAIL_P)])
    pltpu.sync_copy(ea_buf0.at[pl.ds(0, TAIL_P)],
                    acc_ea.at[dsti.at[pl.ds(NFULL_P * CH, TAIL_P)]], add=True)
    plsc.subcore_barrier()

    pltpu.sync_copy(acc_ea.at[pl.ds(s * RPT, RPT)], st16)

    @pl.when(s < NS - 1)
    def _():
        pltpu.sync_copy(
            st16, out_ea.at[pl.ds(s * RPT, RPT), pl.ds(c * DE, DE)])

    @pl.when(s == NS - 1)
    def _():
        pltpu.sync_copy(
            st16.at[pl.ds(0, NV)],
            out_ea.at[pl.ds((NS - 1) * RPT, NV), pl.ds(c * DE, DE)])


@functools.lru_cache(maxsize=None)
def _ea_pass():
    return pl.kernel(
        _ea_body,
        out_type=jax.ShapeDtypeStruct((N, 2 * DE), jnp.float32),
        mesh=_mesh(),
        scratch_types=[
            pltpu.VMEM((EPW,), jnp.int32),
            pltpu.VMEM((CH, DE), jnp.float32),
            pltpu.VMEM((CH, DE), jnp.float32),
            pltpu.VMEM((RPT, DE), jnp.float32),
            pltpu.VMEM_SHARED((NPAD, DE), jnp.float32),
            pltpu.SemaphoreType.DMA,
            pltpu.SemaphoreType.DMA,
        ],
        compiler_params=pltpu.CompilerParams(use_tc_tiling_on_sc=False),
    )


def _edge_body(t_hbm, src2_hbm, dst_hbm, zero_hbm, out_p,
               srci, dsti, rows0, rows1, st_buf, acc, sem0, sem1):
    c = lax.axis_index("c")
    s = lax.axis_index("s")

    pltpu.sync_copy(zero_hbm, st_buf)
    pltpu.sync_copy(st_buf, acc.at[pl.ds(s * RPT, ZR)])
    pltpu.sync_copy(st_buf, acc.at[pl.ds(s * RPT + ZR, ZR)])
    pltpu.sync_copy(src2_hbm.at[c, pl.ds(s * EPT, EPT)], srci)
    pltpu.sync_copy(dst_hbm.at[pl.ds(s * EPT, EPT)], dsti)
    plsc.subcore_barrier()

    pltpu.async_copy(t_hbm.at[srci.at[pl.ds(0, CH)]], rows0, sem0)
    pltpu.async_copy(t_hbm.at[srci.at[pl.ds(CH, CH)]], rows1, sem1)

    def body(i, carry):
        for b, (rows, sem) in enumerate(((rows0, sem0), (rows1, sem1))):
            jj = 2 * i + b
            pltpu.make_async_copy(
                t_hbm.at[srci.at[pl.ds(jj * CH, CH)]], rows, sem).wait()
            pltpu.sync_copy(rows, acc.at[dsti.at[pl.ds(jj * CH, CH)]],
                            add=True)

            @pl.when(jj + 2 < NFULL_E)
            def _():
                pltpu.async_copy(
                    t_hbm.at[srci.at[pl.ds((jj + 2) * CH, CH)]], rows, sem)
        return carry

    lax.fori_loop(0, NFULL_E // 2, body, 0)

    pltpu.async_copy(
        t_hbm.at[srci.at[pl.ds(NFULL_E * CH, TAIL_E)]],
        rows0.at[pl.ds(0, TAIL_E)], sem0)
    pltpu.make_async_copy(
        t_hbm.at[srci.at[pl.ds(NFULL_E * CH, TAIL_E)]],
        rows0.at[pl.ds(0, TAIL_E)], sem0).wait()
    pltpu.sync_copy(rows0.at[pl.ds(0, TAIL_E)],
                    acc.at[dsti.at[pl.ds(NFULL_E * CH, TAIL_E)]], add=True)
    plsc.subcore_barrier()

    pltpu.sync_copy(acc.at[pl.ds(s * RPT, ZR)], st_buf)
    pltpu.sync_copy(
        st_buf, out_p.at[pl.ds(s * RPT, ZR), pl.ds(c * DH, DH)])
    pltpu.sync_copy(acc.at[pl.ds(s * RPT + ZR, ZR)], st_buf)

    @pl.when(s < NS - 1)
    def _():
        pltpu.sync_copy(
            st_buf, out_p.at[pl.ds(s * RPT + ZR, ZR), pl.ds(c * DH, DH)])

    @pl.when(s == NS - 1)
    def _():
        pltpu.sync_copy(
            st_buf.at[pl.ds(0, TAIL)],
            out_p.at[pl.ds((NS - 1) * RPT + ZR, TAIL), pl.ds(c * DH, DH)])


@functools.lru_cache(maxsize=None)
def _edge_pass():
    return pl.kernel(
        _edge_body,
        out_type=jax.ShapeDtypeStruct((N, D), jnp.float32),
        mesh=_mesh(),
        scratch_types=[
            pltpu.VMEM((EPT,), jnp.int32),
            pltpu.VMEM((EPT,), jnp.int32),
            pltpu.VMEM((CH, DH), jnp.float32),
            pltpu.VMEM((CH, DH), jnp.float32),
            pltpu.VMEM((ZR, DH), jnp.float32),
            pltpu.VMEM_SHARED((NPAD, DH), jnp.float32),
            pltpu.SemaphoreType.DMA,
            pltpu.SemaphoreType.DMA,
        ],
        compiler_params=pltpu.CompilerParams(use_tc_tiling_on_sc=False),
    )


_BLK = 1000
_GRID = N // _BLK


def _r_from_dg(dg):
    deg = dg[:, 0:1] + dg[:, DDG:DDG + 1]
    return lax.rsqrt(jnp.maximum(deg, 1.0))


def _tc_pre_body(x_ref, w_ref, dg_ref, t_ref):
    r = _r_from_dg(dg_ref[...])
    t_ref[...] = jnp.dot(x_ref[...], w_ref[...],
                         preferred_element_type=jnp.float32) * r


def _tc_mid_body(p_ref, ea_ref, dg_ref, w_ref, we1_ref, b1_ref, we2_ref,
                 b2_ref, t_ref, base2_ref):
    r = _r_from_dg(dg_ref[...])
    ea = ea_ref[:, 0:DE] + ea_ref[:, DE:2 * DE]
    base1 = jnp.dot(ea, we1_ref[...],
                    preferred_element_type=jnp.float32) + b1_ref[...]
    h = _leaky(r * p_ref[...] + base1)
    t_ref[...] = jnp.dot(h, w_ref[...], preferred_element_type=jnp.float32) * r
    base2_ref[...] = jnp.dot(ea, we2_ref[...],
                             preferred_element_type=jnp.float32) + b2_ref[...]


def _tc_post_body(p_ref, base_ref, dg_ref, out_ref):
    r = _r_from_dg(dg_ref[...])
    out_ref[...] = _leaky(r * p_ref[...] + base_ref[...])


def _idx_body(ei_ref, out_ref):
    sv = ei_ref[0]
    out_ref[0] = 2 * sv
    out_ref[1] = 2 * sv + 1


_EBLK = E // 10

_tc_idx = pl.pallas_call(
    _idx_body,
    grid=(10,),
    in_specs=[pl.BlockSpec((2, _EBLK), lambda i: (0, i))],
    out_specs=pl.BlockSpec((2, _EBLK), lambda i: (0, i)),
    out_shape=jax.ShapeDtypeStruct((2, E), jnp.int32),
)


def _row_spec(width):
    return pl.BlockSpec((_BLK, width), lambda i: (i, 0))


def _full_spec(a, b):
    return pl.BlockSpec((a, b), lambda i: (0, 0))


_f32 = jnp.float32
_sds = jax.ShapeDtypeStruct

_tc_pre = pl.pallas_call(
    _tc_pre_body,
    grid=(_GRID,),
    in_specs=[_row_spec(D), _full_spec(D, D), _row_spec(2 * DDG)],
    out_specs=_row_spec(D),
    out_shape=_sds((N, D), _f32),
)

_tc_mid = pl.pallas_call(
    _tc_mid_body,
    grid=(_GRID,),
    in_specs=[_row_spec(D), _row_spec(2 * DE), _row_spec(2 * DDG),
              _full_spec(D, D), _full_spec(DE, D), _full_spec(1, D),
              _full_spec(DE, D), _full_spec(1, D)],
    out_specs=(_row_spec(D), _row_spec(D)),
    out_shape=(_sds((N, D), _f32), _sds((N, D), _f32)),
)

_tc_post = pl.pallas_call(
    _tc_post_body,
    grid=(_GRID,),
    in_specs=[_row_spec(D), _row_spec(D), _row_spec(2 * DDG)],
    out_specs=_row_spec(D),
    out_shape=_sds((N, D), _f32),
)


def kernel(x, edge_index, edge_attr, W1, We1, b1, W2, We2, b2):
    dst = edge_index[1]
    src2 = _tc_idx(edge_index)
    ones8 = jnp.ones((CH, DDG), _f32)
    z16 = jnp.zeros((RPT, DE), _f32)
    z8 = jnp.zeros((RPT, DDG), _f32)
    z_dh = jnp.zeros((ZR, DH), _f32)

    dg = _deg_pass()(dst, ones8, z8)
    t1 = _tc_pre(x, W1, dg)
    p1 = _edge_pass()(t1.reshape(2 * N, DH), src2, dst, z_dh)
    ea = _ea_pass()(edge_attr, dst, z16, p1)
    t2, base2 = _tc_mid(p1, ea, dg, W2, We1, b1.reshape(1, D),
                        We2, b2.reshape(1, D))
    p2 = _edge_pass()(t2.reshape(2 * N, DH), src2, dst, z_dh)
    return _tc_post(p2, base2, dg)
